# Initial kernel scaffold; baseline (speedup 1.0000x reference)
import jax
import jax.numpy as jnp
from jax.experimental import pallas as pl


def kernel(latent, indices, gru_W_ih, gru_W_hh, gru_b_ih, gru_b_hh, mean_W, mean_b, logvar_W, logvar_b, lstm0_W_ih, lstm0_W_hh, lstm0_b_ih, lstm0_b_hh, lstm1_W_ih, lstm1_W_hh, lstm1_b_ih, lstm1_b_hh, ang_W, ang_b):
    raise NotImplementedError("write your pallas kernel here")


# speedup vs baseline: 46.3080x; 46.3080x over previous
import functools

import jax
import jax.numpy as jnp
from jax import lax
from jax.experimental import pallas as pl
from jax.experimental.pallas import tpu as pltpu
from jax.experimental.pallas import tpu_sc as plsc

B = 16
T = 4096
IN = 128
HID = 256
C = 128
NCH = T // C

_f32 = jnp.float32


def _sigmoid(x):
    return jax.nn.sigmoid(x)


def _gru_cell(x, h, wih, whh, bih, bhh):
    gi = jnp.dot(x, wih, preferred_element_type=_f32) + bih
    gh = jnp.dot(h, whh, preferred_element_type=_f32) + bhh
    r = _sigmoid(gi[:, :IN] + gh[:, :IN])
    z = _sigmoid(gi[:, IN:2 * IN] + gh[:, IN:2 * IN])
    n = jnp.tanh(gi[:, 2 * IN:] + r * gh[:, 2 * IN:])
    return (1.0 - z) * n + z * h


def _lstm_step(g, c):
    i = _sigmoid(g[:, :HID])
    f = _sigmoid(g[:, HID:2 * HID])
    gg = jnp.tanh(g[:, 2 * HID:3 * HID])
    o = _sigmoid(g[:, 3 * HID:])
    c_new = f * c + i * gg
    h_new = o * jnp.tanh(c_new)
    return h_new, c_new


def _phase1_body(js_ref, noise_ref, rm_ref, lat_ref,
                 gwih_ref, gwhh_ref, gbih_ref, gbhh_ref,
                 mwt_ref, mb_ref, lvwt_ref, lvb_ref,
                 l0wih_ref, l0whh_ref, l0bih_ref, l0bhh_ref,
                 x_ref, f0_ref,
                 out_s, g0_s, g1_s, g2_s, lh_s, lc_s):
    j = pl.program_id(0)

    @pl.when(j == 0)
    def _init():
        out_s[...] = jnp.zeros_like(out_s)
        g0_s[...] = jnp.zeros_like(g0_s)
        g1_s[...] = jnp.zeros_like(g1_s)
        g2_s[...] = jnp.zeros_like(g2_s)
        lh_s[...] = jnp.zeros_like(lh_s)
        lc_s[...] = jnp.zeros_like(lc_s)

    @pl.when(j >= js_ref[0])
    def _active():
        lat = lat_ref[...]
        l0bih = l0bih_ref[...]
        l0bhh = l0bhh_ref[...]

        def step(tl, carry):
            out, g0, g1, g2, lh, lc = carry
            mb = rm_ref[tl][:, 0:1] > 0.5
            out = jnp.where(mb, lat, out)
            g0 = jnp.where(mb, 0.0, g0)
            g1 = jnp.where(mb, 0.0, g1)
            g2 = jnp.where(mb, 0.0, g2)
            lh = jnp.where(mb, 0.0, lh)
            lc = jnp.where(mb, 0.0, lc)
            s0 = _gru_cell(out, g0, gwih_ref[0], gwhh_ref[0],
                           gbih_ref[0], gbhh_ref[0])
            s1 = _gru_cell(s0, g1, gwih_ref[1], gwhh_ref[1],
                           gbih_ref[1], gbhh_ref[1])
            s2 = _gru_cell(s1, g2, gwih_ref[2], gwhh_ref[2],
                           gbih_ref[2], gbhh_ref[2])
            mean = jnp.dot(s2, mwt_ref[...], preferred_element_type=_f32) \
                + mb_ref[...]
            std = jnp.exp((jnp.dot(s2, lvwt_ref[...],
                                   preferred_element_type=_f32)
                           + lvb_ref[...]) * 0.5)
            out_new = noise_ref[tl] * std + mean
            x_ref[tl] = out_new
            g = jnp.dot(out_new, l0wih_ref[...],
                        preferred_element_type=_f32) + l0bih \
                + jnp.dot(lh, l0whh_ref[...], preferred_element_type=_f32) \
                + l0bhh
            lh_new, lc_new = _lstm_step(g, lc)
            f0_ref[tl] = lh_new
            return out_new, s0, s1, s2, lh_new, lc_new

        carry = (out_s[...], g0_s[...], g1_s[...], g2_s[...],
                 lh_s[...], lc_s[...])
        carry = lax.fori_loop(0, C, step, carry)
        out_s[...], g0_s[...], g1_s[...], g2_s[...], lh_s[...], lc_s[...] = \
            carry


def _phase2_body(js_ref, x_ref, f0_ref,
                 l0wih_ref, l0whh_ref, l0bih_ref, l0bhh_ref,
                 l1wf_ref, l1wb_ref, l1whh_ref, l1bih_ref, l1bhh_ref,
                 b0_ref, b1_ref,
                 h0_s, c0_s, h1_s, c1_s):
    j = pl.program_id(0)

    @pl.when(j == 0)
    def _init():
        h0_s[...] = jnp.zeros_like(h0_s)
        c0_s[...] = jnp.zeros_like(c0_s)
        h1_s[...] = jnp.zeros_like(h1_s)
        c1_s[...] = jnp.zeros_like(c1_s)

    @pl.when(NCH - 1 - j >= js_ref[0])
    def _active():
        l0bih = l0bih_ref[...]
        l0bhh = l0bhh_ref[...]
        l1bih = l1bih_ref[...]
        l1bhh = l1bhh_ref[...]

        def step(i, carry):
            tl = C - 1 - i
            h0, c0, h1, c1 = carry
            x = x_ref[tl]
            g = jnp.dot(x, l0wih_ref[...],
                        preferred_element_type=_f32) + l0bih \
                + jnp.dot(h0, l0whh_ref[...], preferred_element_type=_f32) \
                + l0bhh
            h0_new, c0_new = _lstm_step(g, c0)
            b0_ref[tl] = h0_new
            g1 = jnp.dot(f0_ref[tl], l1wf_ref[...],
                         preferred_element_type=_f32) \
                + jnp.dot(h0_new, l1wb_ref[...], preferred_element_type=_f32) \
                + l1bih \
                + jnp.dot(h1, l1whh_ref[...], preferred_element_type=_f32) \
                + l1bhh
            h1_new, c1_new = _lstm_step(g1, c1)
            b1_ref[tl] = h1_new
            return h0_new, c0_new, h1_new, c1_new

        carry = (h0_s[...], c0_s[...], h1_s[...], c1_s[...])
        carry = lax.fori_loop(0, C, step, carry)
        h0_s[...], c0_s[...], h1_s[...], c1_s[...] = carry


def _phase3_body(js_ref, f0_ref, b0_ref, b1_ref, rm_ref,
                 l1wf_ref, l1wb_ref, l1whh_ref, l1bih_ref, l1bhh_ref,
                 awf_ref, awb_ref, ab_ref,
                 ang_ref,
                 h_s, c_s):
    j = pl.program_id(0)

    @pl.when(j == 0)
    def _init():
        h_s[...] = jnp.zeros_like(h_s)
        c_s[...] = jnp.zeros_like(c_s)

    @pl.when(j >= js_ref[0])
    def _active():
        l1bih = l1bih_ref[...]
        l1bhh = l1bhh_ref[...]
        ab = ab_ref[...]

        def step(tl, carry):
            h, c = carry
            mb = rm_ref[tl][:, 0:1] > 0.5
            h = jnp.where(mb, 0.0, h)
            c = jnp.where(mb, 0.0, c)
            g = jnp.dot(f0_ref[tl], l1wf_ref[...],
                        preferred_element_type=_f32) \
                + jnp.dot(b0_ref[tl], l1wb_ref[...],
                          preferred_element_type=_f32) \
                + l1bih \
                + jnp.dot(h, l1whh_ref[...], preferred_element_type=_f32) \
                + l1bhh
            h_new, c_new = _lstm_step(g, c)
            raw = jnp.dot(h_new, awf_ref[...], preferred_element_type=_f32) \
                + jnp.dot(b1_ref[tl], awb_ref[...],
                          preferred_element_type=_f32) + ab
            ang = jnp.arctan2(raw[:, :8], raw[:, 8:])
            ang_ref[tl] = jnp.concatenate(
                [ang, jnp.zeros((B, 120), _f32)], axis=1)
            return h_new, c_new

        carry = lax.fori_loop(0, C, step, (h_s[...], c_s[...]))
        h_s[...], c_s[...] = carry


def _chunk_spec(shape, ascending):
    if ascending:
        def imap(j, js):
            return (jnp.maximum(j, js[0]),) + (0,) * (len(shape) - 1)
    else:
        def imap(j, js):
            return (jnp.maximum(NCH - 1 - j, js[0]),) + (0,) * (len(shape) - 1)
    return pl.BlockSpec((C,) + shape[1:], imap)


def _full_spec(shape):
    return pl.BlockSpec(shape, lambda j, js: (0,) * len(shape))


_NW = 32
_TOK_PER_W = T // _NW


def _sc_gather_call(table, gidx):
    mesh = plsc.VectorSubcoreMesh(core_axis_name="c", subcore_axis_name="s")

    @functools.partial(
        pl.kernel, mesh=mesh,
        out_type=jax.ShapeDtypeStruct((T, 128), _f32),
        scratch_types=[
            pltpu.VMEM((_TOK_PER_W,), jnp.int32),
            pltpu.VMEM((_TOK_PER_W, 128), _f32),
            pltpu.SemaphoreType.DMA,
        ],
    )
    def gather_kernel(table_hbm, idx_hbm, out_hbm, idx_v, rows_v, sem):
        wid = lax.axis_index("s") * 2 + lax.axis_index("c")
        base = wid * _TOK_PER_W
        pltpu.sync_copy(idx_hbm.at[pl.ds(base, _TOK_PER_W)], idx_v)
        pltpu.async_copy(table_hbm.at[idx_v], rows_v, sem).wait()
        pltpu.sync_copy(rows_v, out_hbm.at[pl.ds(base, _TOK_PER_W)])

    return gather_kernel(table, gidx)


def kernel(latent, indices, gru_W_ih, gru_W_hh, gru_b_ih, gru_b_hh,
           mean_W, mean_b, logvar_W, logvar_b,
           lstm0_W_ih, lstm0_W_hh, lstm0_b_ih, lstm0_b_hh,
           lstm1_W_ih, lstm1_W_hh, lstm1_b_ih, lstm1_b_hh,
           ang_W, ang_b):
    indices = indices.astype(jnp.int32)

    lengths = jnp.bincount(indices, length=B).astype(jnp.int32)
    starts = jnp.concatenate(
        [jnp.zeros((1,), jnp.int32), jnp.cumsum(lengths)[:-1]])
    pos = jnp.arange(T, dtype=jnp.int32) - starts[indices]
    s = (T - lengths).astype(jnp.int32)
    maxlen = jnp.max(lengths)
    jstart = ((T - maxlen) // C).astype(jnp.int32)[None]

    tgrid = jnp.arange(T, dtype=jnp.int32)[:, None]
    rm = (tgrid == s[None, :]).astype(_f32)
    rm = jnp.repeat(rm[:, :, None], 8, axis=2)

    noise_key = jax.random.key(42)
    noise = jax.vmap(
        lambda t: jax.random.normal(jax.random.fold_in(noise_key, t),
                                    (B, IN), _f32))(jnp.arange(T))
    src_t = jnp.clip(tgrid - s[None, :], 0, T - 1)
    noise_r = jnp.take_along_axis(noise, src_t[:, :, None], axis=0)

    gwih = jnp.swapaxes(gru_W_ih, 1, 2)
    gwhh = jnp.swapaxes(gru_W_hh, 1, 2)
    gbih = gru_b_ih[:, None, :]
    gbhh = gru_b_hh[:, None, :]
    mwt = mean_W.T
    mb2 = mean_b[None, :]
    lvwt = logvar_W.T
    lvb2 = logvar_b[None, :]
    l0wih_f = lstm0_W_ih[0].T
    l0whh_f = lstm0_W_hh[0].T
    l0bih_f = lstm0_b_ih[0][None, :]
    l0bhh_f = lstm0_b_hh[0][None, :]
    l0wih_b = lstm0_W_ih[1].T
    l0whh_b = lstm0_W_hh[1].T
    l0bih_b = lstm0_b_ih[1][None, :]
    l0bhh_b = lstm0_b_hh[1][None, :]
    l1wf_f = lstm1_W_ih[0][:, :HID].T
    l1wb_f = lstm1_W_ih[0][:, HID:].T
    l1whh_f = lstm1_W_hh[0].T
    l1bih_f = lstm1_b_ih[0][None, :]
    l1bhh_f = lstm1_b_hh[0][None, :]
    l1wf_b = lstm1_W_ih[1][:, :HID].T
    l1wb_b = lstm1_W_ih[1][:, HID:].T
    l1whh_b = lstm1_W_hh[1].T
    l1bih_b = lstm1_b_ih[1][None, :]
    l1bhh_b = lstm1_b_hh[1][None, :]

    perm = jnp.array([0, 2, 4], jnp.int32)
    awt = jnp.zeros((2 * HID, 16), _f32)
    awt = awt.at[:, 0:3].set(ang_W[perm].T)
    awt = awt.at[:, 8:11].set(ang_W[perm + 1].T)
    awf = awt[:HID]
    awb = awt[HID:]
    ab = jnp.zeros((1, 16), _f32)
    ab = ab.at[0, 0:3].set(ang_b[perm])
    ab = ab.at[0, 8:11].set(ang_b[perm + 1])

    x_seq, f0 = pl.pallas_call(
        _phase1_body,
        grid_spec=pltpu.PrefetchScalarGridSpec(
            num_scalar_prefetch=1,
            grid=(NCH,),
            in_specs=[
                _chunk_spec((T, B, IN), True),
                _chunk_spec((T, B, 8), True),
                _full_spec((B, IN)),
                _full_spec((3, IN, 3 * IN)),
                _full_spec((3, IN, 3 * IN)),
                _full_spec((3, 1, 3 * IN)),
                _full_spec((3, 1, 3 * IN)),
                _full_spec((IN, IN)),
                _full_spec((1, IN)),
                _full_spec((IN, IN)),
                _full_spec((1, IN)),
                _full_spec((IN, 4 * HID)),
                _full_spec((HID, 4 * HID)),
                _full_spec((1, 4 * HID)),
                _full_spec((1, 4 * HID)),
            ],
            out_specs=[
                _chunk_spec((T, B, IN), True),
                _chunk_spec((T, B, HID), True),
            ],
            scratch_shapes=[
                pltpu.VMEM((B, IN), _f32), pltpu.VMEM((B, IN), _f32),
                pltpu.VMEM((B, IN), _f32), pltpu.VMEM((B, IN), _f32),
                pltpu.VMEM((B, HID), _f32), pltpu.VMEM((B, HID), _f32),
            ],
        ),
        out_shape=[
            jax.ShapeDtypeStruct((T, B, IN), _f32),
            jax.ShapeDtypeStruct((T, B, HID), _f32),
        ],
    )(jstart, noise_r, rm, latent, gwih, gwhh, gbih, gbhh,
      mwt, mb2, lvwt, lvb2, l0wih_f, l0whh_f, l0bih_f, l0bhh_f)

    b0, b1 = pl.pallas_call(
        _phase2_body,
        grid_spec=pltpu.PrefetchScalarGridSpec(
            num_scalar_prefetch=1,
            grid=(NCH,),
            in_specs=[
                _chunk_spec((T, B, IN), False),
                _chunk_spec((T, B, HID), False),
                _full_spec((IN, 4 * HID)),
                _full_spec((HID, 4 * HID)),
                _full_spec((1, 4 * HID)),
                _full_spec((1, 4 * HID)),
                _full_spec((HID, 4 * HID)),
                _full_spec((HID, 4 * HID)),
                _full_spec((HID, 4 * HID)),
                _full_spec((1, 4 * HID)),
                _full_spec((1, 4 * HID)),
            ],
            out_specs=[
                _chunk_spec((T, B, HID), False),
                _chunk_spec((T, B, HID), False),
            ],
            scratch_shapes=[
                pltpu.VMEM((B, HID), _f32), pltpu.VMEM((B, HID), _f32),
                pltpu.VMEM((B, HID), _f32), pltpu.VMEM((B, HID), _f32),
            ],
        ),
        out_shape=[
            jax.ShapeDtypeStruct((T, B, HID), _f32),
            jax.ShapeDtypeStruct((T, B, HID), _f32),
        ],
    )(jstart, x_seq, f0, l0wih_b, l0whh_b, l0bih_b, l0bhh_b,
      l1wf_b, l1wb_b, l1whh_b, l1bih_b, l1bhh_b)

    ang = pl.pallas_call(
        _phase3_body,
        grid_spec=pltpu.PrefetchScalarGridSpec(
            num_scalar_prefetch=1,
            grid=(NCH,),
            in_specs=[
                _chunk_spec((T, B, HID), True),
                _chunk_spec((T, B, HID), True),
                _chunk_spec((T, B, HID), True),
                _chunk_spec((T, B, 8), True),
                _full_spec((HID, 4 * HID)),
                _full_spec((HID, 4 * HID)),
                _full_spec((HID, 4 * HID)),
                _full_spec((1, 4 * HID)),
                _full_spec((1, 4 * HID)),
                _full_spec((HID, 16)),
                _full_spec((HID, 16)),
                _full_spec((1, 16)),
            ],
            out_specs=[_chunk_spec((T, B, 128), True)],
            scratch_shapes=[
                pltpu.VMEM((B, HID), _f32), pltpu.VMEM((B, HID), _f32),
            ],
        ),
        out_shape=[jax.ShapeDtypeStruct((T, B, 128), _f32)],
    )(jstart, f0, b0, b1, rm, l1wf_f, l1wb_f, l1whh_f,
      l1bih_f, l1bhh_f, awf, awb, ab)[0]

    tk = s[indices] + pos
    gidx = tk * B + indices
    table = ang.reshape(T * B, 128)
    gathered = _sc_gather_call(table, gidx)
    return gathered[:, :3]
